# baseline (speedup 1.0000x reference)
import functools

import jax
import jax.numpy as jnp
from jax import lax
from jax.experimental import pallas as pl
from jax.experimental.pallas import tpu as pltpu
from jax.experimental.pallas import tpu_sc as plsc

N = 10000
E = 320000
D = 128
G = 256
MID = 256
NC, NS = 2, 16
NW = NC * NS
CH = 128
CPT0 = 160
CPT1 = 0
TOTCH = NS * (CPT0 + CPT1)
CPT = TOTCH // NW
IBLK = 16
EP = TOTCH * CH
NP = 10240
STR = NP // NS
BR = 400
NB = N // BR

_mesh = functools.partial(
    plsc.VectorSubcoreMesh,
    core_axis_name="c", subcore_axis_name="s", num_cores=NC, num_subcores=NS)


def _hist_body(dstr, zeros1, out, dst_v, ones_v, hist_sh):
    c = lax.axis_index("c")
    s = lax.axis_index("s")
    w = c * NS + s
    off = pl.multiple_of(s * STR, 128)
    pltpu.sync_copy(dstr.at[pl.ds(pl.multiple_of(w * CPT, 8), CPT)], dst_v)
    pltpu.sync_copy(zeros1.at[pl.ds(off, STR)],
                    hist_sh.at[pl.ds(off, STR)])
    for k in range(CH // 16):
        ones_v[pl.ds(k * 16, 16)] = jnp.ones((16,), jnp.float32)
    plsc.subcore_barrier()

    def step(j, carry):
        pltpu.sync_copy(ones_v, hist_sh.at[dst_v.at[j]], add=True)
        return carry

    lax.fori_loop(0, CPT, step, 0)
    plsc.subcore_barrier()
    oout = pl.multiple_of(c * NP + s * STR, 128)
    pltpu.sync_copy(hist_sh.at[pl.ds(off, STR)],
                    out.at[pl.ds(oout, STR)])


def _sc_histogram(dstp, zeros1):
    return pl.kernel(
        _hist_body,
        out_type=jax.ShapeDtypeStruct((NC * NP,), jnp.float32),
        mesh=_mesh(),
        scratch_types=[
            pltpu.VMEM((CPT, CH), jnp.int32),
            pltpu.VMEM((CH,), jnp.float32),
            pltpu.VMEM_SHARED((NP,), jnp.float32),
        ],
    )(dstp, zeros1)


def _prop_body(ht, srcr, dstr, zeros2, out,
               src_v, dst_v, rows0, rows1, acc_sh, gsem0, gsem1):
    c = lax.axis_index("c")
    s = lax.axis_index("s")
    off = pl.multiple_of(s * STR, 128)
    pltpu.sync_copy(zeros2.at[pl.ds(off, STR)],
                    acc_sh.at[pl.ds(off, STR)])
    plsc.subcore_barrier()

    def run(cpt, base):
        def sblock(sb, carry):
            cb = pl.multiple_of(base + sb * IBLK, 8)
            pltpu.sync_copy(srcr.at[pl.ds(cb, IBLK)], src_v)
            pltpu.sync_copy(dstr.at[pl.ds(cb, IBLK)], dst_v)
            pltpu.async_copy(ht.at[src_v.at[0]], rows0, gsem0)

            def step(j, carry):
                j0 = 2 * j
                j1 = j0 + 1
                pltpu.async_copy(ht.at[src_v.at[j1]], rows1, gsem1)
                pltpu.make_async_copy(ht.at[src_v.at[j0]], rows0, gsem0).wait()
                pltpu.sync_copy(rows0, acc_sh.at[dst_v.at[j0]], add=True)

                @pl.when(j < IBLK // 2 - 1)
                def _():
                    pltpu.async_copy(ht.at[src_v.at[j0 + 2]], rows0, gsem0)

                pltpu.make_async_copy(ht.at[src_v.at[j1]], rows1, gsem1).wait()
                pltpu.sync_copy(rows1, acc_sh.at[dst_v.at[j1]], add=True)
                return carry

            return lax.fori_loop(0, IBLK // 2, step, carry)

        lax.fori_loop(0, cpt // IBLK, sblock, 0)

    @pl.when(c == 0)
    def _():
        run(CPT0, s * CPT0)

    @pl.when(c == 1)
    def _():
        run(CPT1, NS * CPT0 + s * CPT1)

    plsc.subcore_barrier()
    pltpu.sync_copy(acc_sh.at[pl.ds(off, STR)],
                    out.at[c, pl.ds(off, STR)])


def _sc_propagate(ht, srcp, dstp, zeros2):
    return pl.kernel(
        _prop_body,
        out_type=jax.ShapeDtypeStruct((NC, NP, D), jnp.float32),
        mesh=_mesh(),
        scratch_types=[
            pltpu.VMEM((IBLK, CH), jnp.int32),
            pltpu.VMEM((IBLK, CH), jnp.int32),
            pltpu.VMEM((CH, D), jnp.float32),
            pltpu.VMEM((CH, D), jnp.float32),
            pltpu.VMEM_SHARED((NP, D), jnp.float32),
            pltpu.SemaphoreType.DMA,
            pltpu.SemaphoreType.DMA,
        ],
    )(ht, srcp, dstp, zeros2)


def _scale_body(x_ref, d_ref, o_ref):
    o_ref[...] = x_ref[...] * d_ref[...]


def _tc_scale(x, dinv2):
    return pl.pallas_call(
        _scale_body,
        grid=(NB,),
        in_specs=[pl.BlockSpec((BR, D), lambda i: (i, 0)),
                  pl.BlockSpec((BR, D), lambda i: (i, 0))],
        out_specs=pl.BlockSpec((BR, D), lambda i: (i, 0)),
        out_shape=jax.ShapeDtypeStruct((N, D), jnp.float32),
    )(x, dinv2)


def _layer_body(last, s_ref, ht_ref, d_ref, w_ref, b_ref, o_ref):
    d = d_ref[...]
    t = (s_ref[0] + s_ref[1] + ht_ref[...]) * d
    o = jnp.dot(t, w_ref[...], preferred_element_type=jnp.float32) + b_ref[...]
    if last:
        o_ref[...] = o
    else:
        o_ref[...] = jnp.maximum(o, 0.0) * d


def _tc_layer(S, ht, dinv2, W, b, last):
    return pl.pallas_call(
        functools.partial(_layer_body, last),
        grid=(NB,),
        in_specs=[pl.BlockSpec((NC, BR, D), lambda i: (0, i, 0)),
                  pl.BlockSpec((BR, D), lambda i: (i, 0)),
                  pl.BlockSpec((BR, D), lambda i: (i, 0)),
                  pl.BlockSpec((D, D), lambda i: (0, 0)),
                  pl.BlockSpec((1, D), lambda i: (0, 0))],
        out_specs=pl.BlockSpec((BR, D), lambda i: (i, 0)),
        out_shape=jax.ShapeDtypeStruct((N, D), jnp.float32),
    )(S, ht, dinv2, W, b)


def _pool_body(bid_ref, h_ref, wm0_ref, bm0_ref, wm1_ref, bm1_ref, z_ref,
               sums, cnt):
    i = pl.program_id(0)

    @pl.when(i == 0)
    def _():
        sums[...] = jnp.zeros((G, D), jnp.float32)
        cnt[...] = jnp.zeros((G, D), jnp.float32)

    ids = bid_ref[0]
    gid = lax.broadcasted_iota(jnp.int32, (G, BR), 0)
    m = jnp.where(ids == gid, 1.0, 0.0)
    sums[...] += jnp.dot(m, h_ref[...], preferred_element_type=jnp.float32)
    cnt[...] += jnp.dot(m, jnp.ones((BR, D), jnp.float32),
                        preferred_element_type=jnp.float32)

    @pl.when(i == NB - 1)
    def _():
        pooled = sums[...] / jnp.maximum(cnt[...], 1.0)
        z1 = jnp.dot(pooled, wm0_ref[...], preferred_element_type=jnp.float32)
        z1 = jnp.maximum(z1 + bm0_ref[...], 0.0)
        z2 = jnp.dot(z1, wm1_ref[...], preferred_element_type=jnp.float32)
        z_ref[...] = jnp.maximum(z2 + bm1_ref[...], 0.0)


def _tc_pool(bid, h2, Wm0, bm0, Wm1, bm1):
    return pl.pallas_call(
        _pool_body,
        grid=(NB,),
        in_specs=[pl.BlockSpec((1, 1, BR), lambda i: (i, 0, 0)),
                  pl.BlockSpec((BR, D), lambda i: (i, 0)),
                  pl.BlockSpec((D, MID), lambda i: (0, 0)),
                  pl.BlockSpec((1, MID), lambda i: (0, 0)),
                  pl.BlockSpec((MID, D), lambda i: (0, 0)),
                  pl.BlockSpec((1, D), lambda i: (0, 0))],
        out_specs=pl.BlockSpec((G, D), lambda i: (0, 0)),
        out_shape=jax.ShapeDtypeStruct((G, D), jnp.float32),
        scratch_shapes=[pltpu.VMEM((G, D), jnp.float32),
                        pltpu.VMEM((G, D), jnp.float32)],
    )(bid, h2, Wm0, bm0, Wm1, bm1)


def kernel(x, edge_index, batch, W0, b0, W1, b1, W2, b2, Wm0, bm0, Wm1, bm1):
    src = edge_index[0]
    dst = edge_index[1]
    pad = EP - E
    srcp = jnp.concatenate([src, jnp.zeros((pad,), jnp.int32)]).reshape(TOTCH, CH)
    srcp = srcp + (jnp.arange(TOTCH, dtype=jnp.int32)[:, None] >= NS * CPT0) * N
    dstp = jnp.concatenate([dst, jnp.full((pad,), N, jnp.int32)]).reshape(TOTCH, CH)
    zeros1 = jnp.zeros((NP,), jnp.float32)
    zeros2 = jnp.zeros((NP, D), jnp.float32)

    counts = _sc_histogram(dstp, zeros1).reshape(NC, NP)
    deg = counts[0, :N] + counts[1, :N] + 1.0
    dinv2 = jnp.broadcast_to(lax.rsqrt(deg)[:, None], (N, D))

    ht = _tc_scale(x, dinv2)
    for W, b, last in ((W0, b0, False), (W1, b1, False), (W2, b2, True)):
        ht2 = jnp.concatenate([ht, ht], axis=0)
        S = _sc_propagate(ht2, srcp, dstp, zeros2)
        ht = _tc_layer(S[:, :N, :], ht, dinv2, W, b.reshape(1, D), last)

    return _tc_pool(batch.reshape(NB, 1, BR), ht,
                    Wm0, bm0.reshape(1, MID), Wm1, bm1.reshape(1, D))


# speedup vs baseline: 2.6866x; 2.6866x over previous
import functools

import jax
import jax.numpy as jnp
from jax import lax
from jax.experimental import pallas as pl
from jax.experimental.pallas import tpu as pltpu
from jax.experimental.pallas import tpu_sc as plsc

N = 10000
E = 320000
D = 128
G = 256
MID = 256
NC, NS = 2, 16
NW = NC * NS
CH = 128
CPT0 = 152
CPT1 = 8
TOTCH = NS * (CPT0 + CPT1)
CPT = TOTCH // NW
IBLK = 16
EP = TOTCH * CH
NP = 10240
STR = NP // NS
BR = 400
NB = N // BR

_mesh = functools.partial(
    plsc.VectorSubcoreMesh,
    core_axis_name="c", subcore_axis_name="s", num_cores=NC, num_subcores=NS)


def _hist_body(dstr, zeros1, out, dst_v, ones_v, hist_sh):
    c = lax.axis_index("c")
    s = lax.axis_index("s")
    w = c * NS + s
    off = pl.multiple_of(s * STR, 128)
    pltpu.sync_copy(dstr.at[pl.ds(pl.multiple_of(w * CPT, 8), CPT)], dst_v)
    pltpu.sync_copy(zeros1.at[pl.ds(off, STR)],
                    hist_sh.at[pl.ds(off, STR)])
    for k in range(CH // 16):
        ones_v[pl.ds(k * 16, 16)] = jnp.ones((16,), jnp.float32)
    plsc.subcore_barrier()

    def step(j, carry):
        pltpu.sync_copy(ones_v, hist_sh.at[dst_v.at[j]], add=True)
        return carry

    lax.fori_loop(0, CPT, step, 0)
    plsc.subcore_barrier()
    oout = pl.multiple_of(c * NP + s * STR, 128)
    pltpu.sync_copy(hist_sh.at[pl.ds(off, STR)],
                    out.at[pl.ds(oout, STR)])


def _sc_histogram(dstp, zeros1):
    return pl.kernel(
        _hist_body,
        out_type=jax.ShapeDtypeStruct((NC * NP,), jnp.float32),
        mesh=_mesh(),
        scratch_types=[
            pltpu.VMEM((CPT, CH), jnp.int32),
            pltpu.VMEM((CH,), jnp.float32),
            pltpu.VMEM_SHARED((NP,), jnp.float32),
        ],
    )(dstp, zeros1)


def _prop_body(ht, srcr, dstr, zeros2, out,
               src_v, dst_v, rows0, rows1, acc_sh, gsem0, gsem1):
    c = lax.axis_index("c")
    s = lax.axis_index("s")
    off = pl.multiple_of(s * STR, 128)
    pltpu.sync_copy(zeros2.at[pl.ds(off, STR)],
                    acc_sh.at[pl.ds(off, STR)])
    plsc.subcore_barrier()

    def run(cpt, base):
        def sblock(sb, carry):
            cb = pl.multiple_of(base + sb * IBLK, 8)
            pltpu.sync_copy(srcr.at[pl.ds(cb, IBLK)], src_v)
            pltpu.sync_copy(dstr.at[pl.ds(cb, IBLK)], dst_v)
            pltpu.async_copy(ht.at[src_v.at[0]], rows0, gsem0)

            def step(j, carry):
                j0 = 2 * j
                j1 = j0 + 1
                pltpu.async_copy(ht.at[src_v.at[j1]], rows1, gsem1)
                pltpu.make_async_copy(ht.at[src_v.at[j0]], rows0, gsem0).wait()
                pltpu.sync_copy(rows0, acc_sh.at[dst_v.at[j0]], add=True)

                @pl.when(j < IBLK // 2 - 1)
                def _():
                    pltpu.async_copy(ht.at[src_v.at[j0 + 2]], rows0, gsem0)

                pltpu.make_async_copy(ht.at[src_v.at[j1]], rows1, gsem1).wait()
                pltpu.sync_copy(rows1, acc_sh.at[dst_v.at[j1]], add=True)
                return carry

            return lax.fori_loop(0, IBLK // 2, step, carry)

        lax.fori_loop(0, cpt // IBLK, sblock, 0)

    @pl.when(c == 0)
    def _():
        run(CPT0, s * CPT0)

    @pl.when(c == 1)
    def _():
        run(CPT1, NS * CPT0 + s * CPT1)

    plsc.subcore_barrier()
    pltpu.sync_copy(acc_sh.at[pl.ds(off, STR)],
                    out.at[c, pl.ds(off, STR)])


def _sc_propagate(ht, srcp, dstp, zeros2):
    return pl.kernel(
        _prop_body,
        out_type=jax.ShapeDtypeStruct((NC, NP, D), jnp.float32),
        mesh=_mesh(),
        scratch_types=[
            pltpu.VMEM((IBLK, CH), jnp.int32),
            pltpu.VMEM((IBLK, CH), jnp.int32),
            pltpu.VMEM((CH, D), jnp.float32),
            pltpu.VMEM((CH, D), jnp.float32),
            pltpu.VMEM_SHARED((NP, D), jnp.float32),
            pltpu.SemaphoreType.DMA,
            pltpu.SemaphoreType.DMA,
        ],
    )(ht, srcp, dstp, zeros2)


def _scale_body(x_ref, d_ref, o_ref):
    o_ref[...] = x_ref[...] * d_ref[...]


def _tc_scale(x, dinv2):
    return pl.pallas_call(
        _scale_body,
        grid=(NB,),
        in_specs=[pl.BlockSpec((BR, D), lambda i: (i, 0)),
                  pl.BlockSpec((BR, D), lambda i: (i, 0))],
        out_specs=pl.BlockSpec((BR, D), lambda i: (i, 0)),
        out_shape=jax.ShapeDtypeStruct((N, D), jnp.float32),
    )(x, dinv2)


def _layer_body(last, s_ref, ht_ref, d_ref, w_ref, b_ref, o_ref):
    d = d_ref[...]
    t = (s_ref[0] + s_ref[1] + ht_ref[...]) * d
    o = jnp.dot(t, w_ref[...], preferred_element_type=jnp.float32) + b_ref[...]
    if last:
        o_ref[...] = o
    else:
        o_ref[...] = jnp.maximum(o, 0.0) * d


def _tc_layer(S, ht, dinv2, W, b, last):
    return pl.pallas_call(
        functools.partial(_layer_body, last),
        grid=(NB,),
        in_specs=[pl.BlockSpec((NC, BR, D), lambda i: (0, i, 0)),
                  pl.BlockSpec((BR, D), lambda i: (i, 0)),
                  pl.BlockSpec((BR, D), lambda i: (i, 0)),
                  pl.BlockSpec((D, D), lambda i: (0, 0)),
                  pl.BlockSpec((1, D), lambda i: (0, 0))],
        out_specs=pl.BlockSpec((BR, D), lambda i: (i, 0)),
        out_shape=jax.ShapeDtypeStruct((N, D), jnp.float32),
    )(S, ht, dinv2, W, b)


def _pool_body(bid_ref, h_ref, wm0_ref, bm0_ref, wm1_ref, bm1_ref, z_ref,
               sums, cnt):
    i = pl.program_id(0)

    @pl.when(i == 0)
    def _():
        sums[...] = jnp.zeros((G, D), jnp.float32)
        cnt[...] = jnp.zeros((G, D), jnp.float32)

    ids = bid_ref[0]
    gid = lax.broadcasted_iota(jnp.int32, (G, BR), 0)
    m = jnp.where(ids == gid, 1.0, 0.0)
    sums[...] += jnp.dot(m, h_ref[...], preferred_element_type=jnp.float32)
    cnt[...] += jnp.dot(m, jnp.ones((BR, D), jnp.float32),
                        preferred_element_type=jnp.float32)

    @pl.when(i == NB - 1)
    def _():
        pooled = sums[...] / jnp.maximum(cnt[...], 1.0)
        z1 = jnp.dot(pooled, wm0_ref[...], preferred_element_type=jnp.float32)
        z1 = jnp.maximum(z1 + bm0_ref[...], 0.0)
        z2 = jnp.dot(z1, wm1_ref[...], preferred_element_type=jnp.float32)
        z_ref[...] = jnp.maximum(z2 + bm1_ref[...], 0.0)


def _tc_pool(bid, h2, Wm0, bm0, Wm1, bm1):
    return pl.pallas_call(
        _pool_body,
        grid=(NB,),
        in_specs=[pl.BlockSpec((1, 1, BR), lambda i: (i, 0, 0)),
                  pl.BlockSpec((BR, D), lambda i: (i, 0)),
                  pl.BlockSpec((D, MID), lambda i: (0, 0)),
                  pl.BlockSpec((1, MID), lambda i: (0, 0)),
                  pl.BlockSpec((MID, D), lambda i: (0, 0)),
                  pl.BlockSpec((1, D), lambda i: (0, 0))],
        out_specs=pl.BlockSpec((G, D), lambda i: (0, 0)),
        out_shape=jax.ShapeDtypeStruct((G, D), jnp.float32),
        scratch_shapes=[pltpu.VMEM((G, D), jnp.float32),
                        pltpu.VMEM((G, D), jnp.float32)],
    )(bid, h2, Wm0, bm0, Wm1, bm1)


def kernel(x, edge_index, batch, W0, b0, W1, b1, W2, b2, Wm0, bm0, Wm1, bm1):
    src = edge_index[0]
    dst = edge_index[1]
    pad = EP - E
    srcp = jnp.concatenate([src, jnp.zeros((pad,), jnp.int32)]).reshape(TOTCH, CH)
    srcp = srcp + (jnp.arange(TOTCH, dtype=jnp.int32)[:, None] >= NS * CPT0) * N
    dstp = jnp.concatenate([dst, jnp.full((pad,), N, jnp.int32)]).reshape(TOTCH, CH)
    zeros1 = jnp.zeros((NP,), jnp.float32)
    zeros2 = jnp.zeros((NP, D), jnp.float32)

    counts = _sc_histogram(dstp, zeros1).reshape(NC, NP)
    deg = counts[0, :N] + counts[1, :N] + 1.0
    dinv2 = jnp.broadcast_to(lax.rsqrt(deg)[:, None], (N, D))

    ht = _tc_scale(x, dinv2)
    for W, b, last in ((W0, b0, False), (W1, b1, False), (W2, b2, True)):
        ht2 = jnp.concatenate([ht, ht], axis=0)
        S = _sc_propagate(ht2, srcp, dstp, zeros2)
        ht = _tc_layer(S[:, :N, :], ht, dinv2, W, b.reshape(1, D), last)

    return _tc_pool(batch.reshape(NB, 1, BR), ht,
                    Wm0, bm0.reshape(1, MID), Wm1, bm1.reshape(1, D))


# speedup vs baseline: 3.8489x; 1.4326x over previous
import functools

import jax
import jax.numpy as jnp
from jax import lax
from jax.experimental import pallas as pl
from jax.experimental.pallas import tpu as pltpu
from jax.experimental.pallas import tpu_sc as plsc

N = 10000
E = 320000
D = 128
G = 256
MID = 256
NC, NS = 2, 16
NW = NC * NS
CH = 128
CPT0 = 80
CPT1 = 80
TOTCH = NS * (CPT0 + CPT1)
CPT = TOTCH // NW
IBLK = 16
EP = TOTCH * CH
NP = 10240
STR = NP // NS
BR = 400
NB = N // BR

_mesh = functools.partial(
    plsc.VectorSubcoreMesh,
    core_axis_name="c", subcore_axis_name="s", num_cores=NC, num_subcores=NS)


def _hist_body(dstr, zeros1, out, dst_v, ones_v, hist_sh):
    c = lax.axis_index("c")
    s = lax.axis_index("s")
    w = c * NS + s
    off = pl.multiple_of(s * STR, 128)
    pltpu.sync_copy(dstr.at[pl.ds(pl.multiple_of(w * CPT, 8), CPT)], dst_v)
    pltpu.sync_copy(zeros1.at[pl.ds(off, STR)],
                    hist_sh.at[pl.ds(off, STR)])
    for k in range(CH // 16):
        ones_v[pl.ds(k * 16, 16)] = jnp.ones((16,), jnp.float32)
    plsc.subcore_barrier()

    def step(j, carry):
        pltpu.sync_copy(ones_v, hist_sh.at[dst_v.at[j]], add=True)
        return carry

    lax.fori_loop(0, CPT, step, 0)
    plsc.subcore_barrier()
    oout = pl.multiple_of(c * NP + s * STR, 128)
    pltpu.sync_copy(hist_sh.at[pl.ds(off, STR)],
                    out.at[pl.ds(oout, STR)])


def _sc_histogram(dstp, zeros1):
    return pl.kernel(
        _hist_body,
        out_type=jax.ShapeDtypeStruct((NC * NP,), jnp.float32),
        mesh=_mesh(),
        scratch_types=[
            pltpu.VMEM((CPT, CH), jnp.int32),
            pltpu.VMEM((CH,), jnp.float32),
            pltpu.VMEM_SHARED((NP,), jnp.float32),
        ],
    )(dstp, zeros1)


def _prop_body(ht, srcr, dstr, zeros2, out,
               src_v, dst_v, rows0, rows1, acc_sh, gsem0, gsem1):
    c = lax.axis_index("c")
    s = lax.axis_index("s")
    off = pl.multiple_of(s * STR, 128)
    pltpu.sync_copy(zeros2.at[pl.ds(off, STR)],
                    acc_sh.at[pl.ds(off, STR)])
    plsc.subcore_barrier()

    def run(cpt, base):
        def sblock(sb, carry):
            cb = pl.multiple_of(base + sb * IBLK, 8)
            pltpu.sync_copy(srcr.at[pl.ds(cb, IBLK)], src_v)
            pltpu.sync_copy(dstr.at[pl.ds(cb, IBLK)], dst_v)
            pltpu.async_copy(ht.at[src_v.at[0]], rows0, gsem0)

            def step(j, carry):
                j0 = 2 * j
                j1 = j0 + 1
                pltpu.async_copy(ht.at[src_v.at[j1]], rows1, gsem1)
                pltpu.make_async_copy(ht.at[src_v.at[j0]], rows0, gsem0).wait()
                pltpu.sync_copy(rows0, acc_sh.at[dst_v.at[j0]], add=True)

                @pl.when(j < IBLK // 2 - 1)
                def _():
                    pltpu.async_copy(ht.at[src_v.at[j0 + 2]], rows0, gsem0)

                pltpu.make_async_copy(ht.at[src_v.at[j1]], rows1, gsem1).wait()
                pltpu.sync_copy(rows1, acc_sh.at[dst_v.at[j1]], add=True)
                return carry

            return lax.fori_loop(0, IBLK // 2, step, carry)

        lax.fori_loop(0, cpt // IBLK, sblock, 0)

    @pl.when(c == 0)
    def _():
        run(CPT0, s * CPT0)

    @pl.when(c == 1)
    def _():
        run(CPT1, NS * CPT0 + s * CPT1)

    plsc.subcore_barrier()
    pltpu.sync_copy(acc_sh.at[pl.ds(off, STR)],
                    out.at[c, pl.ds(off, STR)])


def _sc_propagate(ht, srcp, dstp, zeros2):
    return pl.kernel(
        _prop_body,
        out_type=jax.ShapeDtypeStruct((NC, NP, D), jnp.float32),
        mesh=_mesh(),
        scratch_types=[
            pltpu.VMEM((IBLK, CH), jnp.int32),
            pltpu.VMEM((IBLK, CH), jnp.int32),
            pltpu.VMEM((CH, D), jnp.float32),
            pltpu.VMEM((CH, D), jnp.float32),
            pltpu.VMEM_SHARED((NP, D), jnp.float32),
            pltpu.SemaphoreType.DMA,
            pltpu.SemaphoreType.DMA,
        ],
    )(ht, srcp, dstp, zeros2)


def _scale_body(x_ref, d_ref, o_ref):
    o_ref[...] = x_ref[...] * d_ref[...]


def _tc_scale(x, dinv2):
    return pl.pallas_call(
        _scale_body,
        grid=(NB,),
        in_specs=[pl.BlockSpec((BR, D), lambda i: (i, 0)),
                  pl.BlockSpec((BR, D), lambda i: (i, 0))],
        out_specs=pl.BlockSpec((BR, D), lambda i: (i, 0)),
        out_shape=jax.ShapeDtypeStruct((N, D), jnp.float32),
    )(x, dinv2)


def _layer_body(last, s_ref, ht_ref, d_ref, w_ref, b_ref, o_ref):
    d = d_ref[...]
    t = (s_ref[0] + s_ref[1] + ht_ref[...]) * d
    o = jnp.dot(t, w_ref[...], preferred_element_type=jnp.float32) + b_ref[...]
    if last:
        o_ref[...] = o
    else:
        o_ref[...] = jnp.maximum(o, 0.0) * d


def _tc_layer(S, ht, dinv2, W, b, last):
    return pl.pallas_call(
        functools.partial(_layer_body, last),
        grid=(NB,),
        in_specs=[pl.BlockSpec((NC, BR, D), lambda i: (0, i, 0)),
                  pl.BlockSpec((BR, D), lambda i: (i, 0)),
                  pl.BlockSpec((BR, D), lambda i: (i, 0)),
                  pl.BlockSpec((D, D), lambda i: (0, 0)),
                  pl.BlockSpec((1, D), lambda i: (0, 0))],
        out_specs=pl.BlockSpec((BR, D), lambda i: (i, 0)),
        out_shape=jax.ShapeDtypeStruct((N, D), jnp.float32),
    )(S, ht, dinv2, W, b)


def _pool_body(bid_ref, h_ref, wm0_ref, bm0_ref, wm1_ref, bm1_ref, z_ref,
               sums, cnt):
    i = pl.program_id(0)

    @pl.when(i == 0)
    def _():
        sums[...] = jnp.zeros((G, D), jnp.float32)
        cnt[...] = jnp.zeros((G, D), jnp.float32)

    ids = bid_ref[0]
    gid = lax.broadcasted_iota(jnp.int32, (G, BR), 0)
    m = jnp.where(ids == gid, 1.0, 0.0)
    sums[...] += jnp.dot(m, h_ref[...], preferred_element_type=jnp.float32)
    cnt[...] += jnp.dot(m, jnp.ones((BR, D), jnp.float32),
                        preferred_element_type=jnp.float32)

    @pl.when(i == NB - 1)
    def _():
        pooled = sums[...] / jnp.maximum(cnt[...], 1.0)
        z1 = jnp.dot(pooled, wm0_ref[...], preferred_element_type=jnp.float32)
        z1 = jnp.maximum(z1 + bm0_ref[...], 0.0)
        z2 = jnp.dot(z1, wm1_ref[...], preferred_element_type=jnp.float32)
        z_ref[...] = jnp.maximum(z2 + bm1_ref[...], 0.0)


def _tc_pool(bid, h2, Wm0, bm0, Wm1, bm1):
    return pl.pallas_call(
        _pool_body,
        grid=(NB,),
        in_specs=[pl.BlockSpec((1, 1, BR), lambda i: (i, 0, 0)),
                  pl.BlockSpec((BR, D), lambda i: (i, 0)),
                  pl.BlockSpec((D, MID), lambda i: (0, 0)),
                  pl.BlockSpec((1, MID), lambda i: (0, 0)),
                  pl.BlockSpec((MID, D), lambda i: (0, 0)),
                  pl.BlockSpec((1, D), lambda i: (0, 0))],
        out_specs=pl.BlockSpec((G, D), lambda i: (0, 0)),
        out_shape=jax.ShapeDtypeStruct((G, D), jnp.float32),
        scratch_shapes=[pltpu.VMEM((G, D), jnp.float32),
                        pltpu.VMEM((G, D), jnp.float32)],
    )(bid, h2, Wm0, bm0, Wm1, bm1)


def kernel(x, edge_index, batch, W0, b0, W1, b1, W2, b2, Wm0, bm0, Wm1, bm1):
    src = edge_index[0]
    dst = edge_index[1]
    pad = EP - E
    pad_ids = jnp.arange(pad, dtype=jnp.int32)
    srcp = jnp.concatenate([src, pad_ids % N]).reshape(TOTCH, CH)
    srcp = srcp + (jnp.arange(TOTCH, dtype=jnp.int32)[:, None] >= NS * CPT0) * N
    dstp = jnp.concatenate([dst, N + pad_ids % (NP - N)]).reshape(TOTCH, CH)
    zeros1 = jnp.zeros((NP,), jnp.float32)
    zeros2 = jnp.zeros((NP, D), jnp.float32)

    counts = _sc_histogram(dstp, zeros1).reshape(NC, NP)
    deg = counts[0, :N] + counts[1, :N] + 1.0
    dinv2 = jnp.broadcast_to(lax.rsqrt(deg)[:, None], (N, D))

    ht = _tc_scale(x, dinv2)
    for W, b, last in ((W0, b0, False), (W1, b1, False), (W2, b2, True)):
        ht2 = jnp.concatenate([ht, ht], axis=0)
        S = _sc_propagate(ht2, srcp, dstp, zeros2)
        ht = _tc_layer(S[:, :N, :], ht, dinv2, W, b.reshape(1, D), last)

    return _tc_pool(batch.reshape(NB, 1, BR), ht,
                    Wm0, bm0.reshape(1, MID), Wm1, bm1.reshape(1, D))


# speedup vs baseline: 3.8851x; 1.0094x over previous
import functools

import jax
import jax.numpy as jnp
from jax import lax
from jax.experimental import pallas as pl
from jax.experimental.pallas import tpu as pltpu
from jax.experimental.pallas import tpu_sc as plsc

N = 10000
E = 320000
D = 128
G = 256
MID = 256
NC, NS = 2, 16
NW = NC * NS
CH = 64
CPT = 168
TOTCH = NW * CPT
EP = TOTCH * CH
DBLK = 24
NBUF = 3
NP = 10240
STR = NP // NS
BR = 400
NB = N // BR

_mesh = functools.partial(
    plsc.VectorSubcoreMesh,
    core_axis_name="c", subcore_axis_name="s", num_cores=NC, num_subcores=NS)


def _hist_body(dstr, zeros1, out, dst_v, ones_v, hist_sh):
    c = lax.axis_index("c")
    s = lax.axis_index("s")
    w = c * NS + s
    off = pl.multiple_of(s * STR, 128)
    pltpu.sync_copy(dstr.at[pl.ds(pl.multiple_of(w * CPT, 8), CPT)], dst_v)
    pltpu.sync_copy(zeros1.at[pl.ds(off, STR)],
                    hist_sh.at[pl.ds(off, STR)])
    for k in range(CH // 16):
        ones_v[pl.ds(k * 16, 16)] = jnp.ones((16,), jnp.float32)
    plsc.subcore_barrier()

    def step(j, carry):
        pltpu.sync_copy(ones_v, hist_sh.at[dst_v.at[j]], add=True)
        return carry

    lax.fori_loop(0, CPT, step, 0)
    plsc.subcore_barrier()
    oout = pl.multiple_of(c * NP + s * STR, 128)
    pltpu.sync_copy(hist_sh.at[pl.ds(off, STR)],
                    out.at[pl.ds(oout, STR)])


def _sc_histogram(dstp, zeros1):
    return pl.kernel(
        _hist_body,
        out_type=jax.ShapeDtypeStruct((NC * NP,), jnp.float32),
        mesh=_mesh(),
        scratch_types=[
            pltpu.VMEM((CPT, CH), jnp.int32),
            pltpu.VMEM((CH,), jnp.float32),
            pltpu.VMEM_SHARED((NP,), jnp.float32),
        ],
    )(dstp, zeros1)


def _prop_body(ht, srcr, dstr, zeros2, out,
               src_v, dst_v, b0, b1, b2, acc_sh,
               g0, g1, g2, s0, s1, s2):
    c = lax.axis_index("c")
    s = lax.axis_index("s")
    w = c * NS + s
    off = pl.multiple_of(s * STR, 128)
    base = pl.multiple_of(w * CPT, 8)
    bufs = (b0, b1, b2)
    gsems = (g0, g1, g2)
    ssems = (s0, s1, s2)
    pltpu.sync_copy(srcr.at[pl.ds(base, CPT)], src_v)
    pltpu.sync_copy(zeros2.at[pl.ds(off, STR)],
                    acc_sh.at[pl.ds(off, STR)])
    plsc.subcore_barrier()

    for k in range(NBUF - 1):
        pltpu.async_copy(ht.at[src_v.at[k]], bufs[k], gsems[k])

    def sblock(j, carry):
        @pl.when(j > 0)
        def _():
            pltpu.make_async_copy(bufs[2], acc_sh.at[dst_v.at[0]],
                                  ssems[2]).wait()
        db = pl.multiple_of(base + j * DBLK, 8)
        pltpu.sync_copy(dstr.at[pl.ds(db, DBLK)], dst_v)

        def group(i, carry):
            for k in range(NBUF):
                t = j * DBLK + i * NBUF + k
                local = i * NBUF + k
                pltpu.make_async_copy(ht.at[src_v.at[t]], bufs[k],
                                      gsems[k]).wait()
                if k == 0:
                    @pl.when(i > 0)
                    def _():
                        pltpu.make_async_copy(bufs[2], acc_sh.at[dst_v.at[0]],
                                              ssems[2]).wait()
                else:
                    pltpu.make_async_copy(bufs[k - 1], acc_sh.at[dst_v.at[0]],
                                          ssems[k - 1]).wait()
                pltpu.async_copy(bufs[k], acc_sh.at[dst_v.at[local]],
                                 ssems[k], add=True)

                @pl.when(t + NBUF - 1 < CPT)
                def _(k=k, t=t):
                    pltpu.async_copy(ht.at[src_v.at[t + NBUF - 1]],
                                     bufs[(k + NBUF - 1) % NBUF],
                                     gsems[(k + NBUF - 1) % NBUF])
            return carry

        return lax.fori_loop(0, DBLK // NBUF, group, carry)

    lax.fori_loop(0, CPT // DBLK, sblock, 0)
    pltpu.make_async_copy(bufs[2], acc_sh.at[dst_v.at[0]], ssems[2]).wait()
    plsc.subcore_barrier()
    pltpu.sync_copy(acc_sh.at[pl.ds(off, STR)],
                    out.at[c, pl.ds(off, STR)])


def _sc_propagate(ht, srcp, dstp, zeros2):
    return pl.kernel(
        _prop_body,
        out_type=jax.ShapeDtypeStruct((NC, NP, D), jnp.float32),
        mesh=_mesh(),
        scratch_types=[
            pltpu.VMEM((CPT, CH), jnp.int32),
            pltpu.VMEM((DBLK, CH), jnp.int32),
            pltpu.VMEM((CH, D), jnp.float32),
            pltpu.VMEM((CH, D), jnp.float32),
            pltpu.VMEM((CH, D), jnp.float32),
            pltpu.VMEM_SHARED((NP, D), jnp.float32),
            pltpu.SemaphoreType.DMA,
            pltpu.SemaphoreType.DMA,
            pltpu.SemaphoreType.DMA,
            pltpu.SemaphoreType.DMA,
            pltpu.SemaphoreType.DMA,
            pltpu.SemaphoreType.DMA,
        ],
    )(ht, srcp, dstp, zeros2)


def _scale_body(x_ref, d_ref, o_ref):
    o_ref[...] = x_ref[...] * d_ref[...]


def _tc_scale(x, dinv2):
    return pl.pallas_call(
        _scale_body,
        grid=(NB,),
        in_specs=[pl.BlockSpec((BR, D), lambda i: (i, 0)),
                  pl.BlockSpec((BR, D), lambda i: (i, 0))],
        out_specs=pl.BlockSpec((BR, D), lambda i: (i, 0)),
        out_shape=jax.ShapeDtypeStruct((N, D), jnp.float32),
    )(x, dinv2)


def _layer_body(last, s_ref, ht_ref, d_ref, w_ref, b_ref, o_ref):
    d = d_ref[...]
    t = (s_ref[0] + s_ref[1] + ht_ref[...]) * d
    o = jnp.dot(t, w_ref[...], preferred_element_type=jnp.float32) + b_ref[...]
    if last:
        o_ref[...] = o
    else:
        o_ref[...] = jnp.maximum(o, 0.0) * d


def _tc_layer(S, ht, dinv2, W, b, last):
    return pl.pallas_call(
        functools.partial(_layer_body, last),
        grid=(NB,),
        in_specs=[pl.BlockSpec((NC, BR, D), lambda i: (0, i, 0)),
                  pl.BlockSpec((BR, D), lambda i: (i, 0)),
                  pl.BlockSpec((BR, D), lambda i: (i, 0)),
                  pl.BlockSpec((D, D), lambda i: (0, 0)),
                  pl.BlockSpec((1, D), lambda i: (0, 0))],
        out_specs=pl.BlockSpec((BR, D), lambda i: (i, 0)),
        out_shape=jax.ShapeDtypeStruct((N, D), jnp.float32),
    )(S, ht, dinv2, W, b)


def _pool_body(bid_ref, h_ref, wm0_ref, bm0_ref, wm1_ref, bm1_ref, z_ref,
               sums, cnt):
    i = pl.program_id(0)

    @pl.when(i == 0)
    def _():
        sums[...] = jnp.zeros((G, D), jnp.float32)
        cnt[...] = jnp.zeros((G, D), jnp.float32)

    ids = bid_ref[0]
    gid = lax.broadcasted_iota(jnp.int32, (G, BR), 0)
    m = jnp.where(ids == gid, 1.0, 0.0)
    sums[...] += jnp.dot(m, h_ref[...], preferred_element_type=jnp.float32)
    cnt[...] += jnp.dot(m, jnp.ones((BR, D), jnp.float32),
                        preferred_element_type=jnp.float32)

    @pl.when(i == NB - 1)
    def _():
        pooled = sums[...] / jnp.maximum(cnt[...], 1.0)
        z1 = jnp.dot(pooled, wm0_ref[...], preferred_element_type=jnp.float32)
        z1 = jnp.maximum(z1 + bm0_ref[...], 0.0)
        z2 = jnp.dot(z1, wm1_ref[...], preferred_element_type=jnp.float32)
        z_ref[...] = jnp.maximum(z2 + bm1_ref[...], 0.0)


def _tc_pool(bid, h2, Wm0, bm0, Wm1, bm1):
    return pl.pallas_call(
        _pool_body,
        grid=(NB,),
        in_specs=[pl.BlockSpec((1, 1, BR), lambda i: (i, 0, 0)),
                  pl.BlockSpec((BR, D), lambda i: (i, 0)),
                  pl.BlockSpec((D, MID), lambda i: (0, 0)),
                  pl.BlockSpec((1, MID), lambda i: (0, 0)),
                  pl.BlockSpec((MID, D), lambda i: (0, 0)),
                  pl.BlockSpec((1, D), lambda i: (0, 0))],
        out_specs=pl.BlockSpec((G, D), lambda i: (0, 0)),
        out_shape=jax.ShapeDtypeStruct((G, D), jnp.float32),
        scratch_shapes=[pltpu.VMEM((G, D), jnp.float32),
                        pltpu.VMEM((G, D), jnp.float32)],
    )(bid, h2, Wm0, bm0, Wm1, bm1)


def kernel(x, edge_index, batch, W0, b0, W1, b1, W2, b2, Wm0, bm0, Wm1, bm1):
    src = edge_index[0]
    dst = edge_index[1]
    pad = EP - E
    pad_ids = jnp.arange(pad, dtype=jnp.int32)
    srcp = jnp.concatenate([src, pad_ids % N]).reshape(TOTCH, CH)
    srcp = srcp + (jnp.arange(TOTCH, dtype=jnp.int32)[:, None] >= TOTCH // 2) * N
    dstp = jnp.concatenate([dst, N + pad_ids % (NP - N)]).reshape(TOTCH, CH)
    zeros1 = jnp.zeros((NP,), jnp.float32)
    zeros2 = jnp.zeros((NP, D), jnp.float32)

    counts = _sc_histogram(dstp, zeros1).reshape(NC, NP)
    deg = counts[0, :N] + counts[1, :N] + 1.0
    dinv2 = jnp.broadcast_to(lax.rsqrt(deg)[:, None], (N, D))

    ht = _tc_scale(x, dinv2)
    for W, b, last in ((W0, b0, False), (W1, b1, False), (W2, b2, True)):
        ht2 = jnp.concatenate([ht, ht], axis=0)
        S = _sc_propagate(ht2, srcp, dstp, zeros2)
        ht = _tc_layer(S[:, :N, :], ht, dinv2, W, b.reshape(1, D), last)

    return _tc_pool(batch.reshape(NB, 1, BR), ht,
                    Wm0, bm0.reshape(1, MID), Wm1, bm1.reshape(1, D))


# speedup vs baseline: 4.0993x; 1.0551x over previous
import functools

import jax
import jax.numpy as jnp
from jax import lax
from jax.experimental import pallas as pl
from jax.experimental.pallas import tpu as pltpu
from jax.experimental.pallas import tpu_sc as plsc

N = 10000
E = 320000
D = 128
G = 256
MID = 256
NC, NS = 2, 16
NW = NC * NS
CH = 64
CPT = 168
TOTCH = NW * CPT
EP = TOTCH * CH
DBLK = 24
NBUF = 3
NP = 10240
STR = NP // NS
BR = 400
NB = N // BR

_mesh = functools.partial(
    plsc.VectorSubcoreMesh,
    core_axis_name="c", subcore_axis_name="s", num_cores=NC, num_subcores=NS)


def _hist_body(dstr, zeros1, out, dst_v, ones_v, hist_sh):
    c = lax.axis_index("c")
    s = lax.axis_index("s")
    w = c * NS + s
    off = pl.multiple_of(s * STR, 128)
    pltpu.sync_copy(dstr.at[pl.ds(pl.multiple_of(w * CPT, 8), CPT)], dst_v)
    pltpu.sync_copy(zeros1.at[pl.ds(off, STR)],
                    hist_sh.at[pl.ds(off, STR)])
    for k in range(CH // 16):
        ones_v[pl.ds(k * 16, 16)] = jnp.ones((16,), jnp.float32)
    plsc.subcore_barrier()

    def step(j, carry):
        pltpu.sync_copy(ones_v, hist_sh.at[dst_v.at[j]], add=True)
        return carry

    lax.fori_loop(0, CPT, step, 0)
    plsc.subcore_barrier()
    oout = pl.multiple_of(c * NP + s * STR, 128)
    pltpu.sync_copy(hist_sh.at[pl.ds(off, STR)],
                    out.at[pl.ds(oout, STR)])


def _sc_histogram(dstp, zeros1):
    return pl.kernel(
        _hist_body,
        out_type=jax.ShapeDtypeStruct((NC * NP,), jnp.float32),
        mesh=_mesh(),
        scratch_types=[
            pltpu.VMEM((CPT, CH), jnp.int32),
            pltpu.VMEM((CH,), jnp.float32),
            pltpu.VMEM_SHARED((NP,), jnp.float32),
        ],
    )(dstp, zeros1)


def _prop_body(ht, srcr, dstr, zeros2, out,
               src_v, dst_v, b0, b1, b2, acc_sh,
               g0, g1, g2, s0, s1, s2):
    c = lax.axis_index("c")
    s = lax.axis_index("s")
    w = c * NS + s
    off = pl.multiple_of(s * STR, 128)
    base = pl.multiple_of(w * CPT, 8)
    bufs = (b0, b1, b2)
    gsems = (g0, g1, g2)
    ssems = (s0, s1, s2)
    pltpu.sync_copy(srcr.at[pl.ds(base, CPT)], src_v)
    pltpu.sync_copy(zeros2.at[pl.ds(off, STR)],
                    acc_sh.at[pl.ds(off, STR)])
    plsc.subcore_barrier()

    for k in range(NBUF - 1):
        pltpu.async_copy(ht.at[src_v.at[k]], bufs[k], gsems[k])

    def sblock(j, carry):
        @pl.when(j > 0)
        def _():
            pltpu.make_async_copy(bufs[2], acc_sh.at[dst_v.at[0]],
                                  ssems[2]).wait()
        db = pl.multiple_of(base + j * DBLK, 8)
        pltpu.sync_copy(dstr.at[pl.ds(db, DBLK)], dst_v)

        def group(i, carry):
            for k in range(NBUF):
                t = j * DBLK + i * NBUF + k
                local = i * NBUF + k
                pltpu.make_async_copy(ht.at[src_v.at[t]], bufs[k],
                                      gsems[k]).wait()
                if k == 0:
                    @pl.when(i > 0)
                    def _():
                        pltpu.make_async_copy(bufs[2], acc_sh.at[dst_v.at[0]],
                                              ssems[2]).wait()
                else:
                    pltpu.make_async_copy(bufs[k - 1], acc_sh.at[dst_v.at[0]],
                                          ssems[k - 1]).wait()
                pltpu.async_copy(bufs[k], acc_sh.at[dst_v.at[local]],
                                 ssems[k], add=True)

                @pl.when(t + NBUF - 1 < CPT)
                def _(k=k, t=t):
                    pltpu.async_copy(ht.at[src_v.at[t + NBUF - 1]],
                                     bufs[(k + NBUF - 1) % NBUF],
                                     gsems[(k + NBUF - 1) % NBUF])
            return carry

        return lax.fori_loop(0, DBLK // NBUF, group, carry)

    lax.fori_loop(0, CPT // DBLK, sblock, 0)
    pltpu.make_async_copy(bufs[2], acc_sh.at[dst_v.at[0]], ssems[2]).wait()
    plsc.subcore_barrier()
    pltpu.sync_copy(acc_sh.at[pl.ds(off, STR)],
                    out.at[c, pl.ds(off, STR)])


def _sc_propagate(ht, srcp, dstp, zeros2):
    return pl.kernel(
        _prop_body,
        out_type=jax.ShapeDtypeStruct((NC, NP, D), jnp.float32),
        mesh=_mesh(),
        scratch_types=[
            pltpu.VMEM((CPT, CH), jnp.int32),
            pltpu.VMEM((DBLK, CH), jnp.int32),
            pltpu.VMEM((CH, D), jnp.float32),
            pltpu.VMEM((CH, D), jnp.float32),
            pltpu.VMEM((CH, D), jnp.float32),
            pltpu.VMEM_SHARED((NP, D), jnp.float32),
            pltpu.SemaphoreType.DMA,
            pltpu.SemaphoreType.DMA,
            pltpu.SemaphoreType.DMA,
            pltpu.SemaphoreType.DMA,
            pltpu.SemaphoreType.DMA,
            pltpu.SemaphoreType.DMA,
        ],
    )(ht, srcp, dstp, zeros2)


def _scale_body(x_ref, d_ref, o_ref):
    o_ref[...] = x_ref[...] * d_ref[...]


def _tc_scale(x, dinv2):
    return pl.pallas_call(
        _scale_body,
        grid=(NB,),
        in_specs=[pl.BlockSpec((BR, D), lambda i: (i, 0)),
                  pl.BlockSpec((BR, D), lambda i: (i, 0))],
        out_specs=pl.BlockSpec((BR, D), lambda i: (i, 0)),
        out_shape=jax.ShapeDtypeStruct((N, D), jnp.float32),
    )(x, dinv2)


def _layer_body(last, s_ref, ht_ref, d_ref, w_ref, b_ref, o_ref):
    d = d_ref[...]
    t = (s_ref[0] + s_ref[1] + ht_ref[...]) * d
    o = jnp.dot(t, w_ref[...], preferred_element_type=jnp.float32) + b_ref[...]
    if last:
        o_ref[...] = o
    else:
        o_ref[...] = jnp.maximum(o, 0.0) * d


def _tc_layer(S, ht, dinv2, W, b, last):
    return pl.pallas_call(
        functools.partial(_layer_body, last),
        grid=(NB,),
        in_specs=[pl.BlockSpec((NC, BR, D), lambda i: (0, i, 0)),
                  pl.BlockSpec((BR, D), lambda i: (i, 0)),
                  pl.BlockSpec((BR, D), lambda i: (i, 0)),
                  pl.BlockSpec((D, D), lambda i: (0, 0)),
                  pl.BlockSpec((1, D), lambda i: (0, 0))],
        out_specs=pl.BlockSpec((BR, D), lambda i: (i, 0)),
        out_shape=jax.ShapeDtypeStruct((N, D), jnp.float32),
    )(S, ht, dinv2, W, b)


def _pool_body(bid_ref, h_ref, wm0_ref, bm0_ref, wm1_ref, bm1_ref, z_ref,
               sums, cnt):
    i = pl.program_id(0)

    @pl.when(i == 0)
    def _():
        sums[...] = jnp.zeros((G, D), jnp.float32)
        cnt[...] = jnp.zeros((G, D), jnp.float32)

    ids = bid_ref[0]
    gid = lax.broadcasted_iota(jnp.int32, (G, BR), 0)
    m = jnp.where(ids == gid, 1.0, 0.0)
    sums[...] += jnp.dot(m, h_ref[...], preferred_element_type=jnp.float32)
    cnt[...] += jnp.dot(m, jnp.ones((BR, D), jnp.float32),
                        preferred_element_type=jnp.float32)

    @pl.when(i == NB - 1)
    def _():
        pooled = sums[...] / jnp.maximum(cnt[...], 1.0)
        z1 = jnp.dot(pooled, wm0_ref[...], preferred_element_type=jnp.float32)
        z1 = jnp.maximum(z1 + bm0_ref[...], 0.0)
        z2 = jnp.dot(z1, wm1_ref[...], preferred_element_type=jnp.float32)
        z_ref[...] = jnp.maximum(z2 + bm1_ref[...], 0.0)


def _tc_pool(bid, h2, Wm0, bm0, Wm1, bm1):
    return pl.pallas_call(
        _pool_body,
        grid=(NB,),
        in_specs=[pl.BlockSpec((1, 1, BR), lambda i: (i, 0, 0)),
                  pl.BlockSpec((BR, D), lambda i: (i, 0)),
                  pl.BlockSpec((D, MID), lambda i: (0, 0)),
                  pl.BlockSpec((1, MID), lambda i: (0, 0)),
                  pl.BlockSpec((MID, D), lambda i: (0, 0)),
                  pl.BlockSpec((1, D), lambda i: (0, 0))],
        out_specs=pl.BlockSpec((G, D), lambda i: (0, 0)),
        out_shape=jax.ShapeDtypeStruct((G, D), jnp.float32),
        scratch_shapes=[pltpu.VMEM((G, D), jnp.float32),
                        pltpu.VMEM((G, D), jnp.float32)],
    )(bid, h2, Wm0, bm0, Wm1, bm1)


def kernel(x, edge_index, batch, W0, b0, W1, b1, W2, b2, Wm0, bm0, Wm1, bm1):
    src = edge_index[0]
    dst = edge_index[1]
    pad = EP - E
    pad_ids = jnp.arange(pad, dtype=jnp.int32)
    srcp = jnp.concatenate([src, pad_ids % N]).reshape(TOTCH, CH)
    dstp = jnp.concatenate([dst, N + pad_ids % (NP - N)]).reshape(TOTCH, CH)
    zeros1 = jnp.zeros((NP,), jnp.float32)
    zeros2 = jnp.zeros((NP, D), jnp.float32)

    counts = _sc_histogram(dstp, zeros1).reshape(NC, NP)
    deg = counts[0, :N] + counts[1, :N] + 1.0
    dinv2 = jnp.broadcast_to(lax.rsqrt(deg)[:, None], (N, D))

    ht = _tc_scale(x, dinv2)
    for W, b, last in ((W0, b0, False), (W1, b1, False), (W2, b2, True)):
        S = _sc_propagate(ht, srcp, dstp, zeros2)
        ht = _tc_layer(S[:, :N, :], ht, dinv2, W, b.reshape(1, D), last)

    return _tc_pool(batch.reshape(NB, 1, BR), ht,
                    Wm0, bm0.reshape(1, MID), Wm1, bm1.reshape(1, D))


# speedup vs baseline: 4.2162x; 1.0285x over previous
import functools

import jax
import jax.numpy as jnp
from jax import lax
from jax.experimental import pallas as pl
from jax.experimental.pallas import tpu as pltpu
from jax.experimental.pallas import tpu_sc as plsc

N = 10000
E = 320000
D = 128
G = 256
MID = 256
NC, NS = 2, 16
NW = NC * NS
CH = 64
CPT = 168
TOTCH = NW * CPT
EP = TOTCH * CH
DBLK = 24
NBUF = 3
NP = 10240
STR = NP // NS
BR = 400
NB = N // BR

_mesh = functools.partial(
    plsc.VectorSubcoreMesh,
    core_axis_name="c", subcore_axis_name="s", num_cores=NC, num_subcores=NS)


def _hist_body(dstr, zeros1, out, dst_v, ones_v, hist_sh):
    c = lax.axis_index("c")
    s = lax.axis_index("s")
    w = c * NS + s
    off = pl.multiple_of(s * STR, 128)
    pltpu.sync_copy(dstr.at[pl.ds(pl.multiple_of(w * CPT, 8), CPT)], dst_v)
    pltpu.sync_copy(zeros1.at[pl.ds(off, STR)],
                    hist_sh.at[pl.ds(off, STR)])
    for k in range(CH // 16):
        ones_v[pl.ds(k * 16, 16)] = jnp.ones((16,), jnp.float32)
    plsc.subcore_barrier()

    def step(j, carry):
        pltpu.sync_copy(ones_v, hist_sh.at[dst_v.at[j]], add=True)
        return carry

    lax.fori_loop(0, CPT, step, 0)
    plsc.subcore_barrier()
    oout = pl.multiple_of(c * NP + s * STR, 128)
    pltpu.sync_copy(hist_sh.at[pl.ds(off, STR)],
                    out.at[pl.ds(oout, STR)])


def _sc_histogram(dstp, zeros1):
    return pl.kernel(
        _hist_body,
        out_type=jax.ShapeDtypeStruct((NC * NP,), jnp.float32),
        mesh=_mesh(),
        scratch_types=[
            pltpu.VMEM((CPT, CH), jnp.int32),
            pltpu.VMEM((CH,), jnp.float32),
            pltpu.VMEM_SHARED((NP,), jnp.float32),
        ],
    )(dstp, zeros1)


def _prop_body(ht, srcr, dstr, zeros2, out,
               src_v, dst_v, b0, b1, b2, acc_sh,
               g0, g1, g2, s0, s1, s2):
    c = lax.axis_index("c")
    s = lax.axis_index("s")
    w = c * NS + s
    off = pl.multiple_of(s * STR, 128)
    base = pl.multiple_of(w * CPT, 8)
    bufs = (b0, b1, b2)
    gsems = (g0, g1, g2)
    ssems = (s0, s1, s2)
    pltpu.sync_copy(srcr.at[pl.ds(base, CPT)], src_v)
    pltpu.sync_copy(zeros2.at[pl.ds(off, STR)],
                    acc_sh.at[pl.ds(off, STR)])
    plsc.subcore_barrier()

    for k in range(NBUF - 1):
        pltpu.async_copy(ht.at[src_v.at[k]], bufs[k], gsems[k])

    def sblock(j, carry):
        @pl.when(j > 0)
        def _():
            pltpu.make_async_copy(bufs[2], acc_sh.at[dst_v.at[0]],
                                  ssems[2]).wait()
        db = pl.multiple_of(base + j * DBLK, 8)
        pltpu.sync_copy(dstr.at[pl.ds(db, DBLK)], dst_v)

        def group(i, carry):
            for k in range(NBUF):
                t = j * DBLK + i * NBUF + k
                local = i * NBUF + k
                pltpu.make_async_copy(ht.at[src_v.at[t]], bufs[k],
                                      gsems[k]).wait()
                if k == 0:
                    @pl.when(i > 0)
                    def _():
                        pltpu.make_async_copy(bufs[2], acc_sh.at[dst_v.at[0]],
                                              ssems[2]).wait()
                else:
                    pltpu.make_async_copy(bufs[k - 1], acc_sh.at[dst_v.at[0]],
                                          ssems[k - 1]).wait()
                pltpu.async_copy(bufs[k], acc_sh.at[dst_v.at[local]],
                                 ssems[k], add=True)

                @pl.when(t + NBUF - 1 < CPT)
                def _(k=k, t=t):
                    pltpu.async_copy(ht.at[src_v.at[t + NBUF - 1]],
                                     bufs[(k + NBUF - 1) % NBUF],
                                     gsems[(k + NBUF - 1) % NBUF])
            return carry

        return lax.fori_loop(0, DBLK // NBUF, group, carry)

    lax.fori_loop(0, CPT // DBLK, sblock, 0)
    pltpu.make_async_copy(bufs[2], acc_sh.at[dst_v.at[0]], ssems[2]).wait()
    plsc.subcore_barrier()
    pltpu.sync_copy(acc_sh.at[pl.ds(off, STR)],
                    out.at[c, pl.ds(off, STR)])


def _sc_propagate(ht, srcp, dstp, zeros2):
    return pl.kernel(
        _prop_body,
        out_type=jax.ShapeDtypeStruct((NC, NP, D), jnp.float32),
        mesh=_mesh(),
        scratch_types=[
            pltpu.VMEM((CPT, CH), jnp.int32),
            pltpu.VMEM((DBLK, CH), jnp.int32),
            pltpu.VMEM((CH, D), jnp.float32),
            pltpu.VMEM((CH, D), jnp.float32),
            pltpu.VMEM((CH, D), jnp.float32),
            pltpu.VMEM_SHARED((NP, D), jnp.float32),
            pltpu.SemaphoreType.DMA,
            pltpu.SemaphoreType.DMA,
            pltpu.SemaphoreType.DMA,
            pltpu.SemaphoreType.DMA,
            pltpu.SemaphoreType.DMA,
            pltpu.SemaphoreType.DMA,
        ],
    )(ht, srcp, dstp, zeros2)


def _scale_body(x_ref, d_ref, o_ref):
    o_ref[...] = x_ref[...] * d_ref[...]


def _tc_scale(x, dinv2):
    return pl.pallas_call(
        _scale_body,
        grid=(NB,),
        in_specs=[pl.BlockSpec((BR, D), lambda i: (i, 0)),
                  pl.BlockSpec((BR, D), lambda i: (i, 0))],
        out_specs=pl.BlockSpec((BR, D), lambda i: (i, 0)),
        out_shape=jax.ShapeDtypeStruct((N, D), jnp.float32),
    )(x, dinv2)


def _layer_body(last, s_ref, ht_ref, d_ref, w_ref, b_ref, o_ref):
    d = d_ref[...]
    t = (s_ref[0] + s_ref[1] + ht_ref[...]) * d
    o = jnp.dot(t, w_ref[...], preferred_element_type=jnp.float32) + b_ref[...]
    if last:
        o_ref[...] = o
    else:
        o_ref[...] = jnp.maximum(o, 0.0) * d


def _tc_layer(S, ht, dinv2, W, b, last):
    return pl.pallas_call(
        functools.partial(_layer_body, last),
        grid=(NB,),
        in_specs=[pl.BlockSpec((NC, BR, D), lambda i: (0, i, 0)),
                  pl.BlockSpec((BR, D), lambda i: (i, 0)),
                  pl.BlockSpec((BR, D), lambda i: (i, 0)),
                  pl.BlockSpec((D, D), lambda i: (0, 0)),
                  pl.BlockSpec((1, D), lambda i: (0, 0))],
        out_specs=pl.BlockSpec((BR, D), lambda i: (i, 0)),
        out_shape=jax.ShapeDtypeStruct((N, D), jnp.float32),
    )(S, ht, dinv2, W, b)


def _pool_body(s_ref, ht_ref, d_ref, w_ref, b_ref, bid_ref,
               wm0_ref, bm0_ref, wm1_ref, bm1_ref, z_ref, sums, cnt):
    i = pl.program_id(0)

    @pl.when(i == 0)
    def _():
        sums[...] = jnp.zeros((G, D), jnp.float32)
        cnt[...] = jnp.zeros((G, D), jnp.float32)

    t = (s_ref[0] + s_ref[1] + ht_ref[...]) * d_ref[...]
    h2 = jnp.dot(t, w_ref[...], preferred_element_type=jnp.float32) + b_ref[...]
    ids = bid_ref[0]
    gid = lax.broadcasted_iota(jnp.int32, (G, BR), 0)
    m = jnp.where(ids == gid, 1.0, 0.0)
    sums[...] += jnp.dot(m, h2, preferred_element_type=jnp.float32)
    cnt[...] += jnp.dot(m, jnp.ones((BR, D), jnp.float32),
                        preferred_element_type=jnp.float32)

    @pl.when(i == NB - 1)
    def _():
        pooled = sums[...] / jnp.maximum(cnt[...], 1.0)
        z1 = jnp.dot(pooled, wm0_ref[...], preferred_element_type=jnp.float32)
        z1 = jnp.maximum(z1 + bm0_ref[...], 0.0)
        z2 = jnp.dot(z1, wm1_ref[...], preferred_element_type=jnp.float32)
        z_ref[...] = jnp.maximum(z2 + bm1_ref[...], 0.0)


def _tc_pool(S, ht, dinv2, W, b, bid, Wm0, bm0, Wm1, bm1):
    return pl.pallas_call(
        _pool_body,
        grid=(NB,),
        in_specs=[pl.BlockSpec((NC, BR, D), lambda i: (0, i, 0)),
                  pl.BlockSpec((BR, D), lambda i: (i, 0)),
                  pl.BlockSpec((BR, D), lambda i: (i, 0)),
                  pl.BlockSpec((D, D), lambda i: (0, 0)),
                  pl.BlockSpec((1, D), lambda i: (0, 0)),
                  pl.BlockSpec((1, 1, BR), lambda i: (i, 0, 0)),
                  pl.BlockSpec((D, MID), lambda i: (0, 0)),
                  pl.BlockSpec((1, MID), lambda i: (0, 0)),
                  pl.BlockSpec((MID, D), lambda i: (0, 0)),
                  pl.BlockSpec((1, D), lambda i: (0, 0))],
        out_specs=pl.BlockSpec((G, D), lambda i: (0, 0)),
        out_shape=jax.ShapeDtypeStruct((G, D), jnp.float32),
        scratch_shapes=[pltpu.VMEM((G, D), jnp.float32),
                        pltpu.VMEM((G, D), jnp.float32)],
    )(S, ht, dinv2, W, b, bid, Wm0, bm0, Wm1, bm1)


def kernel(x, edge_index, batch, W0, b0, W1, b1, W2, b2, Wm0, bm0, Wm1, bm1):
    src = edge_index[0]
    dst = edge_index[1]
    pad = EP - E
    pad_ids = jnp.arange(pad, dtype=jnp.int32)
    srcp = jnp.concatenate([src, pad_ids % N]).reshape(TOTCH, CH)
    dstp = jnp.concatenate([dst, N + pad_ids % (NP - N)]).reshape(TOTCH, CH)
    zeros1 = jnp.zeros((NP,), jnp.float32)
    zeros2 = jnp.zeros((NP, D), jnp.float32)

    counts = _sc_histogram(dstp, zeros1).reshape(NC, NP)
    deg = counts[0, :N] + counts[1, :N] + 1.0
    dinv2 = jnp.broadcast_to(lax.rsqrt(deg)[:, None], (N, D))

    ht = _tc_scale(x, dinv2)
    for W, b in ((W0, b0), (W1, b1)):
        S = _sc_propagate(ht, srcp, dstp, zeros2)
        ht = _tc_layer(S[:, :N, :], ht, dinv2, W, b.reshape(1, D), False)

    S = _sc_propagate(ht, srcp, dstp, zeros2)
    return _tc_pool(S[:, :N, :], ht, dinv2, W2, b2.reshape(1, D),
                    batch.reshape(NB, 1, BR),
                    Wm0, bm0.reshape(1, MID), Wm1, bm1.reshape(1, D))


# speedup vs baseline: 4.4989x; 1.0670x over previous
import functools

import jax
import jax.numpy as jnp
from jax import lax
from jax.experimental import pallas as pl
from jax.experimental.pallas import tpu as pltpu
from jax.experimental.pallas import tpu_sc as plsc

N = 10000
E = 320000
D = 128
G = 256
MID = 256
NC, NS = 2, 16
NW = NC * NS
CH = 64
CPT = 168
TOTCH = NW * CPT
EP = TOTCH * CH
DBLK = 24
NBUF = 3
NP = 10240
STR = NP // NS
BR = 400
NB = N // BR

_mesh = functools.partial(
    plsc.VectorSubcoreMesh,
    core_axis_name="c", subcore_axis_name="s", num_cores=NC, num_subcores=NS)


def _hist_body(dstr, zeros1, out, dst_v, ones_v, hist_sh):
    c = lax.axis_index("c")
    s = lax.axis_index("s")
    w = c * NS + s
    off = pl.multiple_of(s * STR, 128)
    pltpu.sync_copy(dstr.at[pl.ds(pl.multiple_of(w * CPT, 8), CPT)], dst_v)
    pltpu.sync_copy(zeros1.at[pl.ds(off, STR)],
                    hist_sh.at[pl.ds(off, STR)])
    for k in range(CH // 16):
        ones_v[pl.ds(k * 16, 16)] = jnp.ones((16,), jnp.float32)
    plsc.subcore_barrier()

    def step(j, carry):
        pltpu.sync_copy(ones_v, hist_sh.at[dst_v.at[j]], add=True)
        return carry

    lax.fori_loop(0, CPT, step, 0)
    plsc.subcore_barrier()
    oout = pl.multiple_of(c * NP + s * STR, 128)
    pltpu.sync_copy(hist_sh.at[pl.ds(off, STR)],
                    out.at[pl.ds(oout, STR)])


def _sc_histogram(dstp, zeros1):
    return pl.kernel(
        _hist_body,
        out_type=jax.ShapeDtypeStruct((NC * NP,), jnp.float32),
        mesh=_mesh(),
        scratch_types=[
            pltpu.VMEM((CPT, CH), jnp.int32),
            pltpu.VMEM((CH,), jnp.float32),
            pltpu.VMEM_SHARED((NP,), jnp.float32),
        ],
    )(dstp, zeros1)


def _prop_body(ht, srcr, dstr, zeros2, out,
               src_v, dst_v, b0, b1, b2, acc_sh,
               g0, g1, g2, s0, s1, s2):
    c = lax.axis_index("c")
    s = lax.axis_index("s")
    w = c * NS + s
    off = pl.multiple_of(s * STR, 128)
    base = pl.multiple_of(w * CPT, 8)
    bufs = (b0, b1, b2)
    gsems = (g0, g1, g2)
    ssems = (s0, s1, s2)
    pltpu.sync_copy(srcr.at[pl.ds(base, CPT)], src_v)
    pltpu.sync_copy(zeros2.at[pl.ds(off, STR)],
                    acc_sh.at[pl.ds(off, STR)])
    plsc.subcore_barrier()

    for k in range(NBUF - 1):
        pltpu.async_copy(ht.at[src_v.at[k]], bufs[k], gsems[k])

    def sblock(j, carry):
        @pl.when(j > 0)
        def _():
            pltpu.make_async_copy(bufs[2], acc_sh.at[dst_v.at[0]],
                                  ssems[2]).wait()
        db = pl.multiple_of(base + j * DBLK, 8)
        pltpu.sync_copy(dstr.at[pl.ds(db, DBLK)], dst_v)

        def group(i, carry):
            for k in range(NBUF):
                t = j * DBLK + i * NBUF + k
                local = i * NBUF + k
                pltpu.make_async_copy(ht.at[src_v.at[t]], bufs[k],
                                      gsems[k]).wait()
                if k == 0:
                    @pl.when(i > 0)
                    def _():
                        pltpu.make_async_copy(bufs[2], acc_sh.at[dst_v.at[0]],
                                              ssems[2]).wait()
                else:
                    pltpu.make_async_copy(bufs[k - 1], acc_sh.at[dst_v.at[0]],
                                          ssems[k - 1]).wait()
                pltpu.async_copy(bufs[k], acc_sh.at[dst_v.at[local]],
                                 ssems[k], add=True)

                @pl.when(t + NBUF - 1 < CPT)
                def _(k=k, t=t):
                    pltpu.async_copy(ht.at[src_v.at[t + NBUF - 1]],
                                     bufs[(k + NBUF - 1) % NBUF],
                                     gsems[(k + NBUF - 1) % NBUF])
            return carry

        return lax.fori_loop(0, DBLK // NBUF, group, carry)

    lax.fori_loop(0, CPT // DBLK, sblock, 0)
    pltpu.make_async_copy(bufs[2], acc_sh.at[dst_v.at[0]], ssems[2]).wait()
    plsc.subcore_barrier()
    pltpu.sync_copy(acc_sh.at[pl.ds(off, STR)],
                    out.at[c, pl.ds(off, STR)])


def _sc_propagate(ht, srcp, dstp, zeros2):
    return pl.kernel(
        _prop_body,
        out_type=jax.ShapeDtypeStruct((NC, NP, D), jnp.bfloat16),
        compiler_params=pltpu.CompilerParams(use_tc_tiling_on_sc=False),
        mesh=_mesh(),
        scratch_types=[
            pltpu.VMEM((CPT, CH), jnp.int32),
            pltpu.VMEM((DBLK, CH), jnp.int32),
            pltpu.VMEM((CH, D), jnp.bfloat16),
            pltpu.VMEM((CH, D), jnp.bfloat16),
            pltpu.VMEM((CH, D), jnp.bfloat16),
            pltpu.VMEM_SHARED((NP, D), jnp.bfloat16),
            pltpu.SemaphoreType.DMA,
            pltpu.SemaphoreType.DMA,
            pltpu.SemaphoreType.DMA,
            pltpu.SemaphoreType.DMA,
            pltpu.SemaphoreType.DMA,
            pltpu.SemaphoreType.DMA,
        ],
    )(ht, srcp, dstp, zeros2)


def _scale_body(x_ref, d_ref, o_ref):
    o_ref[...] = (x_ref[...] * d_ref[...]).astype(jnp.bfloat16)


def _tc_scale(x, dinv2):
    return pl.pallas_call(
        _scale_body,
        grid=(NB,),
        in_specs=[pl.BlockSpec((BR, D), lambda i: (i, 0)),
                  pl.BlockSpec((BR, D), lambda i: (i, 0))],
        out_specs=pl.BlockSpec((BR, D), lambda i: (i, 0)),
        out_shape=jax.ShapeDtypeStruct((N, D), jnp.bfloat16),
    )(x, dinv2)


def _layer_body(last, s_ref, ht_ref, d_ref, w_ref, b_ref, o_ref):
    d = d_ref[...]
    agg = (s_ref[0] + s_ref[1]).astype(jnp.float32) + ht_ref[...].astype(jnp.float32)
    o = jnp.dot(agg * d, w_ref[...], preferred_element_type=jnp.float32) + b_ref[...]
    if last:
        o_ref[...] = o
    else:
        o_ref[...] = (jnp.maximum(o, 0.0) * d).astype(jnp.bfloat16)


def _tc_layer(S, ht, dinv2, W, b, last):
    return pl.pallas_call(
        functools.partial(_layer_body, last),
        grid=(NB,),
        in_specs=[pl.BlockSpec((NC, BR, D), lambda i: (0, i, 0)),
                  pl.BlockSpec((BR, D), lambda i: (i, 0)),
                  pl.BlockSpec((BR, D), lambda i: (i, 0)),
                  pl.BlockSpec((D, D), lambda i: (0, 0)),
                  pl.BlockSpec((1, D), lambda i: (0, 0))],
        out_specs=pl.BlockSpec((BR, D), lambda i: (i, 0)),
        out_shape=jax.ShapeDtypeStruct((N, D), jnp.bfloat16),
    )(S, ht, dinv2, W, b)


def _pool_body(s_ref, ht_ref, d_ref, w_ref, b_ref, bid_ref,
               wm0_ref, bm0_ref, wm1_ref, bm1_ref, z_ref, sums, cnt):
    i = pl.program_id(0)

    @pl.when(i == 0)
    def _():
        sums[...] = jnp.zeros((G, D), jnp.float32)
        cnt[...] = jnp.zeros((G, D), jnp.float32)

    agg = (s_ref[0] + s_ref[1]).astype(jnp.float32) + ht_ref[...].astype(jnp.float32)
    t = agg * d_ref[...]
    h2 = jnp.dot(t, w_ref[...], preferred_element_type=jnp.float32) + b_ref[...]
    ids = bid_ref[0]
    gid = lax.broadcasted_iota(jnp.int32, (G, BR), 0)
    m = jnp.where(ids == gid, 1.0, 0.0)
    sums[...] += jnp.dot(m, h2, preferred_element_type=jnp.float32)
    cnt[...] += jnp.dot(m, jnp.ones((BR, D), jnp.float32),
                        preferred_element_type=jnp.float32)

    @pl.when(i == NB - 1)
    def _():
        pooled = sums[...] / jnp.maximum(cnt[...], 1.0)
        z1 = jnp.dot(pooled, wm0_ref[...], preferred_element_type=jnp.float32)
        z1 = jnp.maximum(z1 + bm0_ref[...], 0.0)
        z2 = jnp.dot(z1, wm1_ref[...], preferred_element_type=jnp.float32)
        z_ref[...] = jnp.maximum(z2 + bm1_ref[...], 0.0)


def _tc_pool(S, ht, dinv2, W, b, bid, Wm0, bm0, Wm1, bm1):
    return pl.pallas_call(
        _pool_body,
        grid=(NB,),
        in_specs=[pl.BlockSpec((NC, BR, D), lambda i: (0, i, 0)),
                  pl.BlockSpec((BR, D), lambda i: (i, 0)),
                  pl.BlockSpec((BR, D), lambda i: (i, 0)),
                  pl.BlockSpec((D, D), lambda i: (0, 0)),
                  pl.BlockSpec((1, D), lambda i: (0, 0)),
                  pl.BlockSpec((1, 1, BR), lambda i: (i, 0, 0)),
                  pl.BlockSpec((D, MID), lambda i: (0, 0)),
                  pl.BlockSpec((1, MID), lambda i: (0, 0)),
                  pl.BlockSpec((MID, D), lambda i: (0, 0)),
                  pl.BlockSpec((1, D), lambda i: (0, 0))],
        out_specs=pl.BlockSpec((G, D), lambda i: (0, 0)),
        out_shape=jax.ShapeDtypeStruct((G, D), jnp.float32),
        scratch_shapes=[pltpu.VMEM((G, D), jnp.float32),
                        pltpu.VMEM((G, D), jnp.float32)],
    )(S, ht, dinv2, W, b, bid, Wm0, bm0, Wm1, bm1)


def kernel(x, edge_index, batch, W0, b0, W1, b1, W2, b2, Wm0, bm0, Wm1, bm1):
    src = edge_index[0]
    dst = edge_index[1]
    pad = EP - E
    pad_ids = jnp.arange(pad, dtype=jnp.int32)
    srcp = jnp.concatenate([src, pad_ids % N]).reshape(TOTCH, CH)
    dstp = jnp.concatenate([dst, N + pad_ids % (NP - N)]).reshape(TOTCH, CH)
    zeros1 = jnp.zeros((NP,), jnp.float32)
    zeros2 = jnp.zeros((NP, D), jnp.bfloat16)

    counts = _sc_histogram(dstp, zeros1).reshape(NC, NP)
    deg = counts[0, :N] + counts[1, :N] + 1.0
    dinv2 = jnp.broadcast_to(lax.rsqrt(deg)[:, None], (N, D))

    ht = _tc_scale(x, dinv2)
    for W, b in ((W0, b0), (W1, b1)):
        S = _sc_propagate(ht, srcp, dstp, zeros2)
        ht = _tc_layer(S[:, :N, :], ht, dinv2, W, b.reshape(1, D), False)

    S = _sc_propagate(ht, srcp, dstp, zeros2)
    return _tc_pool(S[:, :N, :], ht, dinv2, W2, b2.reshape(1, D),
                    batch.reshape(NB, 1, BR),
                    Wm0, bm0.reshape(1, MID), Wm1, bm1.reshape(1, D))


# speedup vs baseline: 5.5858x; 1.2416x over previous
import functools

import jax
import jax.numpy as jnp
from jax import lax
from jax.experimental import pallas as pl
from jax.experimental.pallas import tpu as pltpu
from jax.experimental.pallas import tpu_sc as plsc

N = 10000
E = 320000
D = 128
G = 256
MID = 256
NC, NS = 2, 16
NW = NC * NS
CH = 128
CPT = 80
TOTCH = NW * CPT
EP = TOTCH * CH
DBLK = 16
NBUF = 4
NP = 10240
STR = NP // NS
BR = 400
NB = N // BR

_mesh = functools.partial(
    plsc.VectorSubcoreMesh,
    core_axis_name="c", subcore_axis_name="s", num_cores=NC, num_subcores=NS)


def _hist_body(dstr, zeros1, out, dst_v, ones_v, hist_sh):
    c = lax.axis_index("c")
    s = lax.axis_index("s")
    w = c * NS + s
    off = pl.multiple_of(s * STR, 128)
    pltpu.sync_copy(dstr.at[pl.ds(pl.multiple_of(w * CPT, 8), CPT)], dst_v)
    pltpu.sync_copy(zeros1.at[pl.ds(off, STR)],
                    hist_sh.at[pl.ds(off, STR)])
    for k in range(CH // 16):
        ones_v[pl.ds(k * 16, 16)] = jnp.ones((16,), jnp.float32)
    plsc.subcore_barrier()

    def step(j, carry):
        pltpu.sync_copy(ones_v, hist_sh.at[dst_v.at[j]], add=True)
        return carry

    lax.fori_loop(0, CPT, step, 0)
    plsc.subcore_barrier()
    oout = pl.multiple_of(c * NP + s * STR, 128)
    pltpu.sync_copy(hist_sh.at[pl.ds(off, STR)],
                    out.at[pl.ds(oout, STR)])


def _sc_histogram(dstp, zeros1):
    return pl.kernel(
        _hist_body,
        out_type=jax.ShapeDtypeStruct((NC * NP,), jnp.float32),
        mesh=_mesh(),
        scratch_types=[
            pltpu.VMEM((CPT, CH), jnp.int32),
            pltpu.VMEM((CH,), jnp.float32),
            pltpu.VMEM_SHARED((NP,), jnp.float32),
        ],
    )(dstp, zeros1)


def _prop_body(ht, srcr, dstr, zeros2, out,
               src_v, dst_v, b0, b1, b2, b3, acc_sh,
               g0, g1, g2, g3, s0, s1, s2, s3):
    c = lax.axis_index("c")
    s = lax.axis_index("s")
    w = c * NS + s
    off = pl.multiple_of(s * STR, 128)
    base = pl.multiple_of(w * CPT, 8)
    bufs = (b0, b1, b2, b3)
    gsems = (g0, g1, g2, g3)
    ssems = (s0, s1, s2, s3)
    pltpu.sync_copy(srcr.at[pl.ds(base, CPT)], src_v)
    pltpu.sync_copy(zeros2.at[pl.ds(off, STR)],
                    acc_sh.at[pl.ds(off, STR)])
    plsc.subcore_barrier()

    for k in range(NBUF - 1):
        pltpu.async_copy(ht.at[src_v.at[k]], bufs[k], gsems[k])

    def sblock(j, carry):
        @pl.when(j > 0)
        def _():
            pltpu.make_async_copy(bufs[NBUF - 1], acc_sh.at[dst_v.at[0]],
                                  ssems[NBUF - 1]).wait()
        db = pl.multiple_of(base + j * DBLK, 8)
        pltpu.sync_copy(dstr.at[pl.ds(db, DBLK)], dst_v)

        def group(i, carry):
            for k in range(NBUF):
                t = j * DBLK + i * NBUF + k
                local = i * NBUF + k
                pltpu.make_async_copy(ht.at[src_v.at[t]], bufs[k],
                                      gsems[k]).wait()
                if k == 0:
                    @pl.when(i > 0)
                    def _():
                        pltpu.make_async_copy(bufs[NBUF - 1],
                                              acc_sh.at[dst_v.at[0]],
                                              ssems[NBUF - 1]).wait()
                else:
                    pltpu.make_async_copy(bufs[k - 1], acc_sh.at[dst_v.at[0]],
                                          ssems[k - 1]).wait()
                pltpu.async_copy(bufs[k], acc_sh.at[dst_v.at[local]],
                                 ssems[k], add=True)

                @pl.when(t + NBUF - 1 < CPT)
                def _(k=k, t=t):
                    pltpu.async_copy(ht.at[src_v.at[t + NBUF - 1]],
                                     bufs[(k + NBUF - 1) % NBUF],
                                     gsems[(k + NBUF - 1) % NBUF])
            return carry

        return lax.fori_loop(0, DBLK // NBUF, group, carry)

    lax.fori_loop(0, CPT // DBLK, sblock, 0)
    pltpu.make_async_copy(bufs[NBUF - 1], acc_sh.at[dst_v.at[0]],
                          ssems[NBUF - 1]).wait()
    plsc.subcore_barrier()
    pltpu.sync_copy(acc_sh.at[pl.ds(off, STR)],
                    out.at[c, pl.ds(off, STR)])


def _sc_propagate(ht, srcp, dstp, zeros2):
    return pl.kernel(
        _prop_body,
        out_type=jax.ShapeDtypeStruct((NC, NP, D), jnp.bfloat16),
        compiler_params=pltpu.CompilerParams(use_tc_tiling_on_sc=False),
        mesh=_mesh(),
        scratch_types=[
            pltpu.VMEM((CPT, CH), jnp.int32),
            pltpu.VMEM((DBLK, CH), jnp.int32),
            pltpu.VMEM((CH, D), jnp.bfloat16),
            pltpu.VMEM((CH, D), jnp.bfloat16),
            pltpu.VMEM((CH, D), jnp.bfloat16),
            pltpu.VMEM((CH, D), jnp.bfloat16),
            pltpu.VMEM_SHARED((NP, D), jnp.bfloat16),
            pltpu.SemaphoreType.DMA,
            pltpu.SemaphoreType.DMA,
            pltpu.SemaphoreType.DMA,
            pltpu.SemaphoreType.DMA,
            pltpu.SemaphoreType.DMA,
            pltpu.SemaphoreType.DMA,
            pltpu.SemaphoreType.DMA,
            pltpu.SemaphoreType.DMA,
        ],
    )(ht, srcp, dstp, zeros2)


def _scale_body(x_ref, d_ref, o_ref):
    o_ref[...] = (x_ref[...] * d_ref[...]).astype(jnp.bfloat16)


def _tc_scale(x, dinv2):
    return pl.pallas_call(
        _scale_body,
        grid=(NB,),
        in_specs=[pl.BlockSpec((BR, D), lambda i: (i, 0)),
                  pl.BlockSpec((BR, D), lambda i: (i, 0))],
        out_specs=pl.BlockSpec((BR, D), lambda i: (i, 0)),
        out_shape=jax.ShapeDtypeStruct((N, D), jnp.bfloat16),
    )(x, dinv2)


def _layer_body(last, s_ref, ht_ref, d_ref, w_ref, b_ref, o_ref):
    d = d_ref[...]
    agg = (s_ref[0] + s_ref[1]).astype(jnp.float32) + ht_ref[...].astype(jnp.float32)
    o = jnp.dot(agg * d, w_ref[...], preferred_element_type=jnp.float32) + b_ref[...]
    if last:
        o_ref[...] = o
    else:
        o_ref[...] = (jnp.maximum(o, 0.0) * d).astype(jnp.bfloat16)


def _tc_layer(S, ht, dinv2, W, b, last):
    return pl.pallas_call(
        functools.partial(_layer_body, last),
        grid=(NB,),
        in_specs=[pl.BlockSpec((NC, BR, D), lambda i: (0, i, 0)),
                  pl.BlockSpec((BR, D), lambda i: (i, 0)),
                  pl.BlockSpec((BR, D), lambda i: (i, 0)),
                  pl.BlockSpec((D, D), lambda i: (0, 0)),
                  pl.BlockSpec((1, D), lambda i: (0, 0))],
        out_specs=pl.BlockSpec((BR, D), lambda i: (i, 0)),
        out_shape=jax.ShapeDtypeStruct((N, D), jnp.bfloat16),
    )(S, ht, dinv2, W, b)


def _pool_body(s_ref, ht_ref, d_ref, w_ref, b_ref, bid_ref,
               wm0_ref, bm0_ref, wm1_ref, bm1_ref, z_ref, sums, cnt):
    i = pl.program_id(0)

    @pl.when(i == 0)
    def _():
        sums[...] = jnp.zeros((G, D), jnp.float32)
        cnt[...] = jnp.zeros((G, D), jnp.float32)

    agg = (s_ref[0] + s_ref[1]).astype(jnp.float32) + ht_ref[...].astype(jnp.float32)
    t = agg * d_ref[...]
    h2 = jnp.dot(t, w_ref[...], preferred_element_type=jnp.float32) + b_ref[...]
    ids = bid_ref[0]
    gid = lax.broadcasted_iota(jnp.int32, (G, BR), 0)
    m = jnp.where(ids == gid, 1.0, 0.0)
    sums[...] += jnp.dot(m, h2, preferred_element_type=jnp.float32)
    cnt[...] += jnp.dot(m, jnp.ones((BR, D), jnp.float32),
                        preferred_element_type=jnp.float32)

    @pl.when(i == NB - 1)
    def _():
        pooled = sums[...] / jnp.maximum(cnt[...], 1.0)
        z1 = jnp.dot(pooled, wm0_ref[...], preferred_element_type=jnp.float32)
        z1 = jnp.maximum(z1 + bm0_ref[...], 0.0)
        z2 = jnp.dot(z1, wm1_ref[...], preferred_element_type=jnp.float32)
        z_ref[...] = jnp.maximum(z2 + bm1_ref[...], 0.0)


def _tc_pool(S, ht, dinv2, W, b, bid, Wm0, bm0, Wm1, bm1):
    return pl.pallas_call(
        _pool_body,
        grid=(NB,),
        in_specs=[pl.BlockSpec((NC, BR, D), lambda i: (0, i, 0)),
                  pl.BlockSpec((BR, D), lambda i: (i, 0)),
                  pl.BlockSpec((BR, D), lambda i: (i, 0)),
                  pl.BlockSpec((D, D), lambda i: (0, 0)),
                  pl.BlockSpec((1, D), lambda i: (0, 0)),
                  pl.BlockSpec((1, 1, BR), lambda i: (i, 0, 0)),
                  pl.BlockSpec((D, MID), lambda i: (0, 0)),
                  pl.BlockSpec((1, MID), lambda i: (0, 0)),
                  pl.BlockSpec((MID, D), lambda i: (0, 0)),
                  pl.BlockSpec((1, D), lambda i: (0, 0))],
        out_specs=pl.BlockSpec((G, D), lambda i: (0, 0)),
        out_shape=jax.ShapeDtypeStruct((G, D), jnp.float32),
        scratch_shapes=[pltpu.VMEM((G, D), jnp.float32),
                        pltpu.VMEM((G, D), jnp.float32)],
    )(S, ht, dinv2, W, b, bid, Wm0, bm0, Wm1, bm1)


def kernel(x, edge_index, batch, W0, b0, W1, b1, W2, b2, Wm0, bm0, Wm1, bm1):
    src = edge_index[0]
    dst = edge_index[1]
    pad = EP - E
    pad_ids = jnp.arange(pad, dtype=jnp.int32)
    srcp = jnp.concatenate([src, pad_ids % N]).reshape(TOTCH, CH)
    dstp = jnp.concatenate([dst, N + pad_ids % (NP - N)]).reshape(TOTCH, CH)
    zeros1 = jnp.zeros((NP,), jnp.float32)
    zeros2 = jnp.zeros((NP, D), jnp.bfloat16)

    counts = _sc_histogram(dstp, zeros1).reshape(NC, NP)
    deg = counts[0, :N] + counts[1, :N] + 1.0
    dinv2 = jnp.broadcast_to(lax.rsqrt(deg)[:, None], (N, D))

    ht = _tc_scale(x, dinv2)
    for W, b in ((W0, b0), (W1, b1)):
        S = _sc_propagate(ht, srcp, dstp, zeros2)
        ht = _tc_layer(S[:, :N, :], ht, dinv2, W, b.reshape(1, D), False)

    S = _sc_propagate(ht, srcp, dstp, zeros2)
    return _tc_pool(S[:, :N, :], ht, dinv2, W2, b2.reshape(1, D),
                    batch.reshape(NB, 1, BR),
                    Wm0, bm0.reshape(1, MID), Wm1, bm1.reshape(1, D))
